# baseline (speedup 1.0000x reference)
import functools

import jax
import jax.numpy as jnp
from jax import lax
from jax.experimental import pallas as pl
from jax.experimental.pallas import tpu as pltpu
from jax.experimental.pallas import tpu_sc as plsc

NUM_VOCAB = 100000
EMBED_DIM = 64
N, S, A = 4096, 50, 3
B = N * S
L = 16

NC = 2
NS = 16
NW = NC * NS
B_PER_W = B // NW
HALF = 128
CHUNK = 2 * HALF
N_CHUNK = B_PER_W // CHUNK
H = N // 2


def _sc_gather(value_sn, table):
    mesh = plsc.VectorSubcoreMesh(core_axis_name="c", subcore_axis_name="s")

    @functools.partial(
        pl.kernel,
        mesh=mesh,
        out_type=jax.ShapeDtypeStruct((B // 2, 2, EMBED_DIM), jnp.float32),
        compiler_params=pltpu.CompilerParams(use_tc_tiling_on_sc=False),
        scratch_types=[
            pltpu.VMEM((2, 2, HALF), jnp.int32),
            pltpu.VMEM((2, 2, HALF, EMBED_DIM), jnp.float32),
            pltpu.SemaphoreType.DMA,
            pltpu.SemaphoreType.DMA,
            pltpu.SemaphoreType.DMA,
            pltpu.SemaphoreType.DMA,
            pltpu.SemaphoreType.DMA,
            pltpu.SemaphoreType.DMA,
        ],
    )
    def gather_kernel(value_hbm, table_hbm, out_hbm, idx_v, rows_v,
                      isem_a, isem_b, gsem_a, gsem_b, osem_a, osem_b):
        wid = lax.axis_index("s") * NC + lax.axis_index("c")
        base = wid * B_PER_W
        isem = (isem_a, isem_b)
        gsem = (gsem_a, gsem_b)
        osem = (osem_a, osem_b)

        def src_starts(j):
            jj0 = base + j * CHUNK
            s = jj0 // N
            cc = (jj0 % N) // 2
            pe0 = pl.multiple_of(s * N + cc, 8)
            po0 = pl.multiple_of(pe0 + H, 8)
            return pe0, po0

        def fire_idx(j, buf):
            pe0, po0 = src_starts(j)
            return [
                pltpu.async_copy(value_hbm.at[pl.ds(pe0, HALF)],
                                 idx_v.at[buf, 0], isem[buf]),
                pltpu.async_copy(value_hbm.at[pl.ds(po0, HALF)],
                                 idx_v.at[buf, 1], isem[buf]),
            ]

        def fire_gathers(j, buf):
            return [
                pltpu.async_copy(table_hbm.at[idx_v.at[buf, h]],
                                 rows_v.at[buf, h], gsem[buf])
                for h in range(2)
            ]

        def fire_out(j, buf):
            rq = pl.multiple_of(base // 2 + j * HALF, 8)
            return [
                pltpu.async_copy(rows_v.at[buf, h],
                                 out_hbm.at[pl.ds(rq, HALF), h], osem[buf])
                for h in range(2)
            ]

        for c in fire_idx(0, 0):
            c.wait()
        pending_g = fire_gathers(0, 0)
        pending_i = fire_idx(1, 1)
        pending_o = [None, None]
        for j in range(N_CHUNK):
            cur = j % 2
            nxt = (j + 1) % 2
            for c in pending_g:
                c.wait()
            if j + 1 < N_CHUNK:
                for c in pending_i:
                    c.wait()
                if pending_o[nxt] is not None:
                    for c in pending_o[nxt]:
                        c.wait()
                    pending_o[nxt] = None
                next_g = fire_gathers(j + 1, nxt)
            pending_o[cur] = fire_out(j, cur)
            if j + 2 < N_CHUNK:
                pending_i = fire_idx(j + 2, cur)
            if j + 1 < N_CHUNK:
                pending_g = next_g
        for po in pending_o:
            if po is not None:
                for c in po:
                    c.wait()

    return gather_kernel(value_sn, table)


H = N // 2


def _tc_add_transpose(g2, p2t, W2, eye):
    RPS = N * EMBED_DIM // 128

    def add_t_kernel(g_ref, p_ref, w_ref, e_ref, o_ref):
        x = g_ref[...]
        lin = lax.dot_general(
            p_ref[0], w_ref[...], (((0,), (0,)), ((), ())),
            preferred_element_type=jnp.float32,
            precision=jax.lax.Precision.HIGHEST)
        del e_ref
        yT = (x + lin).T
        o_ref[0, :, 0:H] = yT[0:EMBED_DIM, :]
        o_ref[0, :, H:N] = yT[EMBED_DIM:128, :]

    return pl.pallas_call(
        add_t_kernel,
        grid=(S,),
        in_specs=[
            pl.BlockSpec((RPS, 128), lambda i: (i, 0)),
            pl.BlockSpec((1, 8, H), lambda i: (i, 0, 0)),
            pl.BlockSpec((8, 128), lambda i: (0, 0)),
            pl.BlockSpec((128, 128), lambda i: (0, 0)),
        ],
        out_specs=pl.BlockSpec((1, EMBED_DIM, N), lambda i: (i, 0, 0)),
        out_shape=jax.ShapeDtypeStruct((S, EMBED_DIM, N), jnp.float32),
    )(g2, p2t, W2, eye)


def kernel(value, depth, position, table, W, b):
    del depth
    value_sn = value.T.reshape(B)
    gathered = _sc_gather(value_sn, table)
    g2 = gathered.reshape(B * EMBED_DIM // 128, 128)
    pos_pl = position.transpose(2, 1, 0)
    ones = jnp.ones((1, S, H), jnp.float32)
    p2t = jnp.concatenate(
        [pos_pl[:, :, :H], ones, pos_pl[:, :, H:], ones],
        axis=0).transpose(1, 0, 2)
    z = jnp.zeros_like(W)
    zb = jnp.zeros_like(b)
    W2 = jnp.concatenate([
        jnp.concatenate([W, z], axis=1),
        jnp.concatenate([b.reshape(1, -1), zb.reshape(1, -1)], axis=1),
        jnp.concatenate([z, W], axis=1),
        jnp.concatenate([zb.reshape(1, -1), b.reshape(1, -1)], axis=1),
    ], axis=0)
    eye = jnp.eye(128, dtype=jnp.float32)
    out_t = _tc_add_transpose(g2, p2t, W2, eye)
    return jnp.transpose(out_t, (2, 0, 1))


# speedup vs baseline: 2.7157x; 2.7157x over previous
import functools

import jax
import jax.numpy as jnp
from jax import lax
from jax.experimental import pallas as pl
from jax.experimental.pallas import tpu as pltpu
from jax.experimental.pallas import tpu_sc as plsc

NUM_VOCAB = 100000
EMBED_DIM = 64
N, S, A = 4096, 50, 3
B = N * S
L = 16

NC = 2
NS = 16
NW = NC * NS
B_PER_W = B // NW
HALF = 128
CHUNK = 2 * HALF
N_CHUNK = B_PER_W // CHUNK
H = N // 2


def _sc_gather(value_sn, table):
    mesh = plsc.VectorSubcoreMesh(core_axis_name="c", subcore_axis_name="s")

    @functools.partial(
        pl.kernel,
        mesh=mesh,
        out_type=jax.ShapeDtypeStruct((B // 2, 2 * EMBED_DIM), jnp.float32),
        compiler_params=pltpu.CompilerParams(use_tc_tiling_on_sc=False),
        scratch_types=[
            pltpu.VMEM((2, 2, HALF), jnp.int32),
            pltpu.VMEM((2, 2, HALF, EMBED_DIM), jnp.float32),
            pltpu.SemaphoreType.DMA,
            pltpu.SemaphoreType.DMA,
            pltpu.SemaphoreType.DMA,
            pltpu.SemaphoreType.DMA,
            pltpu.SemaphoreType.DMA,
            pltpu.SemaphoreType.DMA,
        ],
    )
    def gather_kernel(value_hbm, table_hbm, out_hbm, idx_v, rows_v,
                      isem_a, isem_b, gsem_a, gsem_b, osem_a, osem_b):
        wid = lax.axis_index("s") * NC + lax.axis_index("c")
        base = wid * B_PER_W
        isem = (isem_a, isem_b)
        gsem = (gsem_a, gsem_b)
        osem = (osem_a, osem_b)

        def src_starts(j):
            jj0 = base + j * CHUNK
            s = jj0 // N
            cc = (jj0 % N) // 2
            pe0 = pl.multiple_of(s * N + cc, 8)
            po0 = pl.multiple_of(pe0 + H, 8)
            return pe0, po0

        def fire_idx(j, buf):
            pe0, po0 = src_starts(j)
            return [
                pltpu.async_copy(value_hbm.at[pl.ds(pe0, HALF)],
                                 idx_v.at[buf, 0], isem[buf]),
                pltpu.async_copy(value_hbm.at[pl.ds(po0, HALF)],
                                 idx_v.at[buf, 1], isem[buf]),
            ]

        def fire_gathers(j, buf):
            return [
                pltpu.async_copy(table_hbm.at[idx_v.at[buf, h]],
                                 rows_v.at[buf, h], gsem[buf])
                for h in range(2)
            ]

        def fire_out(j, buf):
            rq = pl.multiple_of(base // 2 + j * HALF, 8)
            return [
                pltpu.async_copy(
                    rows_v.at[buf, h],
                    out_hbm.at[pl.ds(rq, HALF), pl.ds(h * EMBED_DIM, EMBED_DIM)],
                    osem[buf])
                for h in range(2)
            ]

        for c in fire_idx(0, 0):
            c.wait()
        pending_g = fire_gathers(0, 0)
        pending_i = fire_idx(1, 1)
        pending_o = [None, None]
        for j in range(N_CHUNK):
            cur = j % 2
            nxt = (j + 1) % 2
            for c in pending_g:
                c.wait()
            if j + 1 < N_CHUNK:
                for c in pending_i:
                    c.wait()
                if pending_o[nxt] is not None:
                    for c in pending_o[nxt]:
                        c.wait()
                    pending_o[nxt] = None
                next_g = fire_gathers(j + 1, nxt)
            pending_o[cur] = fire_out(j, cur)
            if j + 2 < N_CHUNK:
                pending_i = fire_idx(j + 2, cur)
            if j + 1 < N_CHUNK:
                pending_g = next_g
        for po in pending_o:
            if po is not None:
                for c in po:
                    c.wait()

    return gather_kernel(value_sn, table)


H = N // 2


def _tc_add_transpose(g2, p2t, W2, eye):
    RPS = N * EMBED_DIM // 128

    def add_t_kernel(g_ref, p_ref, w_ref, e_ref, o_ref):
        x = g_ref[...]
        lin = lax.dot_general(
            p_ref[0], w_ref[...], (((0,), (0,)), ((), ())),
            preferred_element_type=jnp.float32,
            precision=jax.lax.Precision.HIGHEST)
        del e_ref
        yT = (x + lin).T
        o_ref[0, :, 0:H] = yT[0:EMBED_DIM, :]
        o_ref[0, :, H:N] = yT[EMBED_DIM:128, :]

    return pl.pallas_call(
        add_t_kernel,
        grid=(S,),
        in_specs=[
            pl.BlockSpec((RPS, 128), lambda i: (i, 0)),
            pl.BlockSpec((1, 8, H), lambda i: (i, 0, 0)),
            pl.BlockSpec((8, 128), lambda i: (0, 0)),
            pl.BlockSpec((128, 128), lambda i: (0, 0)),
        ],
        out_specs=pl.BlockSpec((1, EMBED_DIM, N), lambda i: (i, 0, 0)),
        out_shape=jax.ShapeDtypeStruct((S, EMBED_DIM, N), jnp.float32),
    )(g2, p2t, W2, eye)


def kernel(value, depth, position, table, W, b):
    del depth
    value_sn = value.T.reshape(B)
    g2 = _sc_gather(value_sn, table)
    pos_pl = position.transpose(2, 1, 0)
    ones = jnp.ones((1, S, H), jnp.float32)
    p2t = jnp.concatenate(
        [pos_pl[:, :, :H], ones, pos_pl[:, :, H:], ones],
        axis=0).transpose(1, 0, 2)
    z = jnp.zeros_like(W)
    zb = jnp.zeros_like(b)
    W2 = jnp.concatenate([
        jnp.concatenate([W, z], axis=1),
        jnp.concatenate([b.reshape(1, -1), zb.reshape(1, -1)], axis=1),
        jnp.concatenate([z, W], axis=1),
        jnp.concatenate([zb.reshape(1, -1), b.reshape(1, -1)], axis=1),
    ], axis=0)
    eye = jnp.eye(128, dtype=jnp.float32)
    out_t = _tc_add_transpose(g2, p2t, W2, eye)
    return jnp.transpose(out_t, (2, 0, 1))
